# baseline (speedup 1.0000x reference)
import functools

import jax
import jax.numpy as jnp
from jax import lax
from jax.experimental import pallas as pl
from jax.experimental.pallas import tpu as pltpu
from jax.experimental.pallas import tpu_sc as plsc

L = 16
ROWS = 128
N = 32768
NV = N // L
RANK = 513
NWORKERS = 32
ROWS_PER_W = ROWS // NWORKERS
HIST = 256
HSTRIDE = 257
MIN_I32 = -2147483648
SUF0 = 0
TOT0 = 288
SCRATCH = 576


def _select_digit(hist_ref, scr_ref, lane, r):
    zeros = jnp.zeros((L,), jnp.int32)
    r_v = jnp.full((L,), r, jnp.int32)
    for c in range(16):
        tot = jnp.zeros((L,), jnp.int32)
        for l in range(16):
            tot = tot + hist_ref[pl.ds(l * HSTRIDE + c * L, L)]
            hist_ref[pl.ds(l * HSTRIDE + c * L, L)] = zeros
        suf = lax.rev(jnp.cumsum(lax.rev(tot, (0,))), (0,))
        scr_ref[pl.ds(SUF0 + c * 17, L)] = suf
        scr_ref[pl.ds(TOT0 + c * 17, L)] = tot
    tr = lane * 17
    ctot_v = plsc.load_gather(scr_ref, [tr + SUF0])
    carry_v = lax.rev(jnp.cumsum(lax.rev(ctot_v, (0,))), (0,)) - ctot_v
    cnt_v = jnp.zeros((L,), jnp.int32)
    tota_v = jnp.zeros((L,), jnp.int32)
    keep_v = jnp.zeros((L,), jnp.int32)
    for l in range(16):
        suf_t = plsc.load_gather(scr_ref, [tr + (SUF0 + l)])
        tot_t = plsc.load_gather(scr_ref, [tr + (TOT0 + l)])
        m = (suf_t + carry_v) >= r_v
        cnt_v = cnt_v + plsc.all_reduce_population_count(m)
        keep_v = keep_v + jnp.where(m, tot_t, 0)
        tota_v = tota_v + tot_t
    bstar = jnp.max(cnt_v) - 1
    cab = jnp.sum(tota_v - keep_v)
    return bstar, cab


def _body(in_hbm, out_hbm, x2_v, c_v, hist_v, scr_v, si, so):
    cid = lax.axis_index("c")
    sid = lax.axis_index("s")
    wid = sid * 2 + cid
    lane = lax.iota(jnp.int32, L)
    lane_off = lane * HSTRIDE
    ones = jnp.ones((L,), jnp.int32)
    zeros = jnp.zeros((L,), jnp.int32)
    sign_v = jnp.full((L,), MIN_I32, jnp.int32)
    row0 = wid * ROWS_PER_W

    @plsc.parallel_loop(0, HSTRIDE * 16 // L, unroll=8)
    def zero_hist(j):
        hist_v[pl.ds(j * L, L)] = zeros

    def mono(x):
        v = plsc.bitcast(x, jnp.int32)
        s = lax.shift_right_arithmetic(v, 31)
        return lax.bitwise_xor(v, lax.bitwise_or(s, sign_v))

    def do_row(i, _):
        row = row0 + i
        x_v = x2_v.at[pl.ds((i & 1) * N, N)]
        pltpu.make_async_copy(in_hbm.at[row], x_v, si).wait()

        @plsc.parallel_loop(0, NV, unroll=16)
        def p3(j):
            u = mono(x_v[pl.ds(j * L, L)])
            d = lax.shift_right_logical(u, 24)
            plsc.addupdate_scatter(hist_v, [lane_off + d], ones)

        b, cab = _select_digit(hist_v, scr_v, lane, jnp.int32(RANK))
        prefix = b
        r = jnp.int32(RANK) - cab

        other_v = x2_v.at[pl.ds(((i & 1) ^ 1) * N, N)]

        @pl.when(i > 0)
        def _():
            pltpu.make_async_copy(other_v, out_hbm.at[row - 1], so).wait()

        @pl.when(i < ROWS_PER_W - 1)
        def _():
            pltpu.make_async_copy(in_hbm.at[row + 1], other_v, si).start()

        neg = prefix < 128
        p3raw_v = jnp.full((L,), jnp.where(neg, 255 - prefix, prefix - 128),
                           jnp.int32)
        flip_v = jnp.full((L,), jnp.where(neg, jnp.int32(255), jnp.int32(0)),
                          jnp.int32)

        @plsc.parallel_loop(0, NV, unroll=16,
                            carry=jnp.full((L,), -1, jnp.int32))
        def p2(j, off_v):
            v = plsc.bitcast(x_v[pl.ds(j * L, L)], jnp.int32)
            match = lax.shift_right_logical(v, 24) == p3raw_v
            d = lax.bitwise_xor(
                lax.bitwise_and(lax.shift_right_logical(v, 16), 255), flip_v)
            plsc.addupdate_scatter(hist_v, [lane_off + d], ones, mask=match)
            mi = match.astype(jnp.int32)
            pos = off_v + jnp.cumsum(mi)
            plsc.store_scatter(c_v, [pos], v, mask=match)
            return off_v + plsc.all_reduce_population_count(match)

        cnt_v = p2 + ones
        b, cab = _select_digit(hist_v, scr_v, lane, r)
        prefix = lax.shift_left(prefix, 8) | b
        r = r - cab
        nv2 = (jnp.max(cnt_v) + (L - 1)) >> 4

        pref2_v = jnp.full(
            (L,), prefix ^ jnp.where(neg, jnp.int32(0xFFFF),
                                     jnp.int32(0x8000)), jnp.int32)

        @plsc.parallel_loop(0, nv2, unroll=8)
        def p1(j):
            v = c_v[pl.ds(j * L, L)]
            valid = (j * L + lane) < cnt_v
            match = jnp.logical_and(
                lax.shift_right_logical(v, 16) == pref2_v, valid)
            d = lax.bitwise_xor(
                lax.bitwise_and(lax.shift_right_logical(v, 8), 255), flip_v)
            plsc.addupdate_scatter(hist_v, [lane_off + d], ones, mask=match)

        b, cab = _select_digit(hist_v, scr_v, lane, r)
        prefix = lax.shift_left(prefix, 8) | b
        r = r - cab

        pref1_v = jnp.full(
            (L,), prefix ^ jnp.where(neg, jnp.int32(0xFFFFFF),
                                     jnp.int32(0x800000)), jnp.int32)

        @plsc.parallel_loop(0, nv2, unroll=8)
        def p0(j):
            v = c_v[pl.ds(j * L, L)]
            valid = (j * L + lane) < cnt_v
            match = jnp.logical_and(
                lax.shift_right_logical(v, 8) == pref1_v, valid)
            d = lax.bitwise_xor(lax.bitwise_and(v, 255), flip_v)
            plsc.addupdate_scatter(hist_v, [lane_off + d], ones, mask=match)

        b, _cab = _select_digit(hist_v, scr_v, lane, r)
        prefix = lax.shift_left(prefix, 8) | b

        ut_v = jnp.full((L,), prefix, jnp.int32)
        xmask = lax.bitwise_or(
            lax.bitwise_not(lax.shift_right_arithmetic(ut_v, 31)), sign_v)
        t_v = plsc.bitcast(lax.bitwise_xor(ut_v, xmask), jnp.float32)

        @plsc.parallel_loop(0, NV, unroll=16)
        def mbody(j):
            x = x_v[pl.ds(j * L, L)]
            x_v[pl.ds(j * L, L)] = jnp.where(x > t_v, x, 0.0)

        pltpu.make_async_copy(x_v, out_hbm.at[row], so).start()
        return 0

    pltpu.make_async_copy(in_hbm.at[row0], x2_v.at[pl.ds(0, N)], si).start()
    lax.fori_loop(0, ROWS_PER_W, do_row, 0)
    last = ROWS_PER_W - 1
    pltpu.make_async_copy(
        x2_v.at[pl.ds((last & 1) * N, N)],
        out_hbm.at[row0 + last], so).wait()


@jax.jit
def _ksparse(inputs):
    mesh = plsc.VectorSubcoreMesh(core_axis_name="c", subcore_axis_name="s")
    f = functools.partial(
        pl.kernel,
        mesh=mesh,
        out_type=jax.ShapeDtypeStruct((ROWS, N), jnp.float32),
        compiler_params=pltpu.CompilerParams(needs_layout_passes=False),
        scratch_types=[
            pltpu.VMEM((2 * N,), jnp.float32),
            pltpu.VMEM((N,), jnp.int32),
            pltpu.VMEM((16 * HSTRIDE,), jnp.int32),
            pltpu.VMEM((SCRATCH,), jnp.int32),
            pltpu.SemaphoreType.DMA,
            pltpu.SemaphoreType.DMA,
        ],
    )(_body)
    return f(inputs)


def kernel(inputs):
    return _ksparse(inputs)


# speedup vs baseline: 1.1013x; 1.1013x over previous
import functools

import jax
import jax.numpy as jnp
from jax import lax
from jax.experimental import pallas as pl
from jax.experimental.pallas import tpu as pltpu
from jax.experimental.pallas import tpu_sc as plsc

L = 16
ROWS = 128
N = 32768
NV = N // L
RANK = 513
NWORKERS = 32
ROWS_PER_W = ROWS // NWORKERS
HIST = 256
HSTRIDE = 257
MIN_I32 = -2147483648
SUF0 = 0
TOT0 = 288
SCRATCH = 576


def _select_digit(hist_ref, scr_ref, lane, r):
    zeros = jnp.zeros((L,), jnp.int32)
    r_v = jnp.full((L,), r, jnp.int32)

    @plsc.parallel_loop(0, 16, unroll=2)
    def chunk(c):
        tot = jnp.zeros((L,), jnp.int32)
        for l in range(16):
            tot = tot + hist_ref[pl.ds(l * HSTRIDE + c * L, L)]
            hist_ref[pl.ds(l * HSTRIDE + c * L, L)] = zeros
        suf = lax.rev(jnp.cumsum(lax.rev(tot, (0,))), (0,))
        scr_ref[pl.ds(SUF0 + c * 17, L)] = suf
        scr_ref[pl.ds(TOT0 + c * 17, L)] = tot

    tr = lane * 17
    ctot_v = plsc.load_gather(scr_ref, [tr + SUF0])
    carry_v = lax.rev(jnp.cumsum(lax.rev(ctot_v, (0,))), (0,)) - ctot_v

    def ph3(l, acc):
        cnt_v, tota_v, keep_v = acc
        suf_t = plsc.load_gather(scr_ref, [tr + (SUF0 + l)])
        tot_t = plsc.load_gather(scr_ref, [tr + (TOT0 + l)])
        m = (suf_t + carry_v) >= r_v
        cnt_v = cnt_v + plsc.all_reduce_population_count(m)
        keep_v = keep_v + jnp.where(m, tot_t, 0)
        tota_v = tota_v + tot_t
        return (cnt_v, tota_v, keep_v)

    z = jnp.zeros((L,), jnp.int32)
    cnt_v, tota_v, keep_v = lax.fori_loop(0, 16, ph3, (z, z, z))
    bstar = jnp.max(cnt_v) - 1
    cab = jnp.sum(tota_v - keep_v)
    return bstar, cab


def _body(in_hbm, out_hbm, x2_v, c_v, hist_v, scr_v, si, so):
    cid = lax.axis_index("c")
    sid = lax.axis_index("s")
    wid = sid * 2 + cid
    lane = lax.iota(jnp.int32, L)
    lane_off = lane * HSTRIDE
    ones = jnp.ones((L,), jnp.int32)
    zeros = jnp.zeros((L,), jnp.int32)
    sign_v = jnp.full((L,), MIN_I32, jnp.int32)
    row0 = wid * ROWS_PER_W

    @plsc.parallel_loop(0, HSTRIDE * 16 // L, unroll=8)
    def zero_hist(j):
        hist_v[pl.ds(j * L, L)] = zeros

    def mono(x):
        v = plsc.bitcast(x, jnp.int32)
        s = lax.shift_right_arithmetic(v, 31)
        return lax.bitwise_xor(v, lax.bitwise_or(s, sign_v))

    def do_row(i, _):
        row = row0 + i
        x_v = x2_v.at[pl.ds((i & 1) * N, N)]
        pltpu.make_async_copy(in_hbm.at[row], x_v, si).wait()

        @plsc.parallel_loop(0, NV, unroll=16)
        def p3(j):
            u = mono(x_v[pl.ds(j * L, L)])
            d = lax.shift_right_logical(u, 24)
            plsc.addupdate_scatter(hist_v, [lane_off + d], ones)

        b, cab = _select_digit(hist_v, scr_v, lane, jnp.int32(RANK))
        prefix = b
        r = jnp.int32(RANK) - cab

        other_v = x2_v.at[pl.ds(((i & 1) ^ 1) * N, N)]

        @pl.when(i > 0)
        def _():
            pltpu.make_async_copy(other_v, out_hbm.at[row - 1], so).wait()

        @pl.when(i < ROWS_PER_W - 1)
        def _():
            pltpu.make_async_copy(in_hbm.at[row + 1], other_v, si).start()

        neg = prefix < 128
        p3raw_v = jnp.full((L,), jnp.where(neg, 255 - prefix, prefix - 128),
                           jnp.int32)
        flip_v = jnp.full((L,), jnp.where(neg, jnp.int32(255), jnp.int32(0)),
                          jnp.int32)

        @plsc.parallel_loop(0, NV, unroll=8,
                            carry=jnp.full((L,), -1, jnp.int32))
        def p2(j, off_v):
            v = plsc.bitcast(x_v[pl.ds(j * L, L)], jnp.int32)
            match = lax.shift_right_logical(v, 24) == p3raw_v
            d = lax.bitwise_xor(
                lax.bitwise_and(lax.shift_right_logical(v, 16), 255), flip_v)
            plsc.addupdate_scatter(hist_v, [lane_off + d], ones, mask=match)
            mi = match.astype(jnp.int32)
            pos = off_v + jnp.cumsum(mi)
            plsc.store_scatter(c_v, [pos], v, mask=match)
            return off_v + plsc.all_reduce_population_count(match)

        cnt_v = p2 + ones
        b, cab = _select_digit(hist_v, scr_v, lane, r)
        prefix = lax.shift_left(prefix, 8) | b
        r = r - cab
        nv2 = (jnp.max(cnt_v) + (L - 1)) >> 4

        pref2_v = jnp.full(
            (L,), prefix ^ jnp.where(neg, jnp.int32(0xFFFF),
                                     jnp.int32(0x8000)), jnp.int32)

        @plsc.parallel_loop(0, nv2, unroll=4)
        def p1(j):
            v = c_v[pl.ds(j * L, L)]
            valid = (j * L + lane) < cnt_v
            match = jnp.logical_and(
                lax.shift_right_logical(v, 16) == pref2_v, valid)
            d = lax.bitwise_xor(
                lax.bitwise_and(lax.shift_right_logical(v, 8), 255), flip_v)
            plsc.addupdate_scatter(hist_v, [lane_off + d], ones, mask=match)

        b, cab = _select_digit(hist_v, scr_v, lane, r)
        prefix = lax.shift_left(prefix, 8) | b
        r = r - cab

        pref1_v = jnp.full(
            (L,), prefix ^ jnp.where(neg, jnp.int32(0xFFFFFF),
                                     jnp.int32(0x800000)), jnp.int32)

        @plsc.parallel_loop(0, nv2, unroll=4)
        def p0(j):
            v = c_v[pl.ds(j * L, L)]
            valid = (j * L + lane) < cnt_v
            match = jnp.logical_and(
                lax.shift_right_logical(v, 8) == pref1_v, valid)
            d = lax.bitwise_xor(lax.bitwise_and(v, 255), flip_v)
            plsc.addupdate_scatter(hist_v, [lane_off + d], ones, mask=match)

        b, _cab = _select_digit(hist_v, scr_v, lane, r)
        prefix = lax.shift_left(prefix, 8) | b

        ut_v = jnp.full((L,), prefix, jnp.int32)
        xmask = lax.bitwise_or(
            lax.bitwise_not(lax.shift_right_arithmetic(ut_v, 31)), sign_v)
        t_v = plsc.bitcast(lax.bitwise_xor(ut_v, xmask), jnp.float32)

        @plsc.parallel_loop(0, NV, unroll=16)
        def mbody(j):
            x = x_v[pl.ds(j * L, L)]
            x_v[pl.ds(j * L, L)] = jnp.where(x > t_v, x, 0.0)

        pltpu.make_async_copy(x_v, out_hbm.at[row], so).start()
        return 0

    pltpu.make_async_copy(in_hbm.at[row0], x2_v.at[pl.ds(0, N)], si).start()
    lax.fori_loop(0, ROWS_PER_W, do_row, 0)
    last = ROWS_PER_W - 1
    pltpu.make_async_copy(
        x2_v.at[pl.ds((last & 1) * N, N)],
        out_hbm.at[row0 + last], so).wait()


@jax.jit
def _ksparse(inputs):
    mesh = plsc.VectorSubcoreMesh(core_axis_name="c", subcore_axis_name="s")
    f = functools.partial(
        pl.kernel,
        mesh=mesh,
        out_type=jax.ShapeDtypeStruct((ROWS, N), jnp.float32),
        compiler_params=pltpu.CompilerParams(needs_layout_passes=False),
        scratch_types=[
            pltpu.VMEM((2 * N,), jnp.float32),
            pltpu.VMEM((N,), jnp.int32),
            pltpu.VMEM((16 * HSTRIDE,), jnp.int32),
            pltpu.VMEM((SCRATCH,), jnp.int32),
            pltpu.SemaphoreType.DMA,
            pltpu.SemaphoreType.DMA,
        ],
    )(_body)
    return f(inputs)


def kernel(inputs):
    return _ksparse(inputs)
